# baseline (speedup 1.0000x reference)
import jax
import jax.numpy as jnp
from jax import lax
from jax.experimental import pallas as pl
from jax.experimental.pallas import tpu as pltpu

_LAYERS = 4
_N_OPS = 16
_BLOCK = 4
_GRID = 128 // _BLOCK
_BSPEC = pl.BlockSpec((_BLOCK, 3, 224, 224), lambda i: (i, 0, 0, 0))
_SMEM = pl.BlockSpec(memory_space=pltpu.SMEM)


def _affine_kernel(ab_ref, x_ref, o_ref):
    o_ref[...] = ab_ref[0] * x_ref[...] + ab_ref[1]


def _general_kernel(a_ref, b_ref, c_ref, d_ref, x_ref, o_ref):
    x = x_ref[...]
    for j in range(_LAYERS):
        x = a_ref[j] * x + b_ref[j] + c_ref[j] * jnp.sin(x) \
            + d_ref[j] * jnp.tanh(x)
    o_ref[...] = x


def _run_affine(ab, a, b, c, d, image):
    return pl.pallas_call(
        _affine_kernel,
        grid=(_GRID,),
        in_specs=[_SMEM, _BSPEC],
        out_specs=_BSPEC,
        out_shape=jax.ShapeDtypeStruct(image.shape, jnp.float32),
    )(ab, image)


def _run_general(ab, a, b, c, d, image):
    return pl.pallas_call(
        _general_kernel,
        grid=(_GRID,),
        in_specs=[_SMEM, _SMEM, _SMEM, _SMEM, _BSPEC],
        out_specs=_BSPEC,
        out_shape=jax.ShapeDtypeStruct(image.shape, jnp.float32),
    )(a, b, c, d, image)


def kernel(image, probs_per_layer, magnitudes):
    logits = jnp.log(probs_per_layer + 1e-9)
    u = jax.random.uniform(jax.random.key(42), logits.shape, jnp.float32,
                           minval=jnp.finfo(jnp.float32).tiny, maxval=1.0)
    scores = logits - jnp.log(-jnp.log(u))
    opers = jnp.argmax(scores, axis=-1)
    onehot = (jnp.arange(_N_OPS)[None, :] == opers[:, None]).astype(jnp.float32)
    m0 = jnp.sum(magnitudes[:_LAYERS] * onehot, axis=1)
    m1 = jnp.sum(magnitudes[_LAYERS:] * onehot, axis=1)
    k = opers % 8

    is_sin = k == 4
    is_tanh = k == 6
    is_aff = ~(is_sin | is_tanh)
    a_aff = jnp.where(k == 2, 1.0 + m0,
            jnp.where(k == 3, -1.0,
            jnp.where(k == 5, m1,
            jnp.where(k == 7, 1.0 / (1.0 + jnp.abs(m1)), 1.0))))
    b_aff = jnp.where((k == 1) | (k == 5), m0, jnp.where(k == 3, m1, 0.0))

    a = jnp.where(is_aff, a_aff, jnp.where(is_sin, 1.0, 0.0))
    b = jnp.where(is_aff, b_aff, 0.0)
    c = jnp.where(is_sin, m0, 0.0)
    d = jnp.where(is_tanh, 1.0 + m1, 0.0)

    A = a_aff[0]
    B = b_aff[0]
    for j in range(1, _LAYERS):
        A = a_aff[j] * A
        B = a_aff[j] * B + b_aff[j]
    ab = jnp.stack([A, B])

    args = (ab.astype(jnp.float32), a.astype(jnp.float32),
            b.astype(jnp.float32), c.astype(jnp.float32),
            d.astype(jnp.float32), image)
    return lax.cond(jnp.any(~is_aff), _run_general, _run_affine, *args)


# speedup vs baseline: 1.0179x; 1.0179x over previous
import jax
import jax.numpy as jnp
from jax import lax
from jax.experimental import pallas as pl
from jax.experimental.pallas import tpu as pltpu

_LAYERS = 4
_N_OPS = 16
_BLOCK = 16
_GRID = 128 // _BLOCK
_BSPEC = pl.BlockSpec((_BLOCK, 3, 224, 224), lambda i: (i, 0, 0, 0))
_SMEM = pl.BlockSpec(memory_space=pltpu.SMEM)


def _affine_kernel(ab_ref, x_ref, o_ref):
    o_ref[...] = ab_ref[0] * x_ref[...] + ab_ref[1]


def _general_kernel(a_ref, b_ref, c_ref, d_ref, x_ref, o_ref):
    x = x_ref[...]
    for j in range(_LAYERS):
        x = a_ref[j] * x + b_ref[j] + c_ref[j] * jnp.sin(x) \
            + d_ref[j] * jnp.tanh(x)
    o_ref[...] = x


def _run_affine(ab, a, b, c, d, image):
    return pl.pallas_call(
        _affine_kernel,
        grid=(_GRID,),
        in_specs=[_SMEM, _BSPEC],
        out_specs=_BSPEC,
        out_shape=jax.ShapeDtypeStruct(image.shape, jnp.float32),
    )(ab, image)


def _run_general(ab, a, b, c, d, image):
    return pl.pallas_call(
        _general_kernel,
        grid=(_GRID,),
        in_specs=[_SMEM, _SMEM, _SMEM, _SMEM, _BSPEC],
        out_specs=_BSPEC,
        out_shape=jax.ShapeDtypeStruct(image.shape, jnp.float32),
    )(a, b, c, d, image)


def kernel(image, probs_per_layer, magnitudes):
    logits = jnp.log(probs_per_layer + 1e-9)
    u = jax.random.uniform(jax.random.key(42), logits.shape, jnp.float32,
                           minval=jnp.finfo(jnp.float32).tiny, maxval=1.0)
    scores = logits - jnp.log(-jnp.log(u))
    opers = jnp.argmax(scores, axis=-1)
    onehot = (jnp.arange(_N_OPS)[None, :] == opers[:, None]).astype(jnp.float32)
    m0 = jnp.sum(magnitudes[:_LAYERS] * onehot, axis=1)
    m1 = jnp.sum(magnitudes[_LAYERS:] * onehot, axis=1)
    k = opers % 8

    is_sin = k == 4
    is_tanh = k == 6
    is_aff = ~(is_sin | is_tanh)
    a_aff = jnp.where(k == 2, 1.0 + m0,
            jnp.where(k == 3, -1.0,
            jnp.where(k == 5, m1,
            jnp.where(k == 7, 1.0 / (1.0 + jnp.abs(m1)), 1.0))))
    b_aff = jnp.where((k == 1) | (k == 5), m0, jnp.where(k == 3, m1, 0.0))

    a = jnp.where(is_aff, a_aff, jnp.where(is_sin, 1.0, 0.0))
    b = jnp.where(is_aff, b_aff, 0.0)
    c = jnp.where(is_sin, m0, 0.0)
    d = jnp.where(is_tanh, 1.0 + m1, 0.0)

    A = a_aff[0]
    B = b_aff[0]
    for j in range(1, _LAYERS):
        A = a_aff[j] * A
        B = a_aff[j] * B + b_aff[j]
    ab = jnp.stack([A, B])

    args = (ab.astype(jnp.float32), a.astype(jnp.float32),
            b.astype(jnp.float32), c.astype(jnp.float32),
            d.astype(jnp.float32), image)
    return lax.cond(jnp.any(~is_aff), _run_general, _run_affine, *args)
